# Initial kernel scaffold; baseline (speedup 1.0000x reference)
import jax
import jax.numpy as jnp
from jax.experimental import pallas as pl


def kernel(x, edge_index, cheb_W, cheb_b, fc1_W, fc1_b, bn1_gamma, bn1_beta, bn1_mean, bn1_var, fc2_W, fc2_b, bn2_gamma, bn2_beta, bn2_mean, bn2_var, fc3_W, fc3_b):
    raise NotImplementedError("write your pallas kernel here")


# speedup vs baseline: 13.4485x; 13.4485x over previous
import jax
import jax.numpy as jnp
from jax import lax
from jax.experimental import pallas as pl
from jax.experimental.pallas import tpu as pltpu
from jax.experimental.pallas import tpu_sc as plsc

N = 15872
E = 507904
CIN = 128
HALF = CIN // 2
COUT = 256
NC, NS = 2, 16
CHUNK = 128
RPT = N // NS
ZROWS = 248
BATCH = 256
LIN1 = 512
LIN2 = 256
EPS = 1e-5

_MESH = plsc.VectorSubcoreMesh(
    core_axis_name="c", subcore_axis_name="s", num_cores=NC, num_subcores=NS)

_f32 = jnp.float32


def _sc_degree_body(src_hbm, zeros_hbm, ones_hbm, degp_hbm,
                    acc, zbuf, ones_v, sidx, gsem):
    c = lax.axis_index("c")
    s = lax.axis_index("s")
    r0 = s * RPT
    pltpu.sync_copy(zeros_hbm, zbuf)
    for j in range(RPT // ZROWS):
        pltpu.sync_copy(zbuf, acc.at[pl.ds(r0 + j * ZROWS, ZROWS)])
    pltpu.sync_copy(ones_hbm, ones_v)
    plsc.subcore_barrier()

    w = s * NC + c
    epw = E // (NC * NS)
    nchunks = epw // CHUNK
    ebase = w * epw

    def body(t, carry):
        off = ebase + t * CHUNK
        pltpu.sync_copy(src_hbm.at[pl.ds(off, CHUNK)], sidx)
        pltpu.sync_copy(ones_v, acc.at[sidx], add=True)
        return carry
    lax.fori_loop(0, nchunks, body, 0)

    plsc.subcore_barrier()
    for j in range(RPT // ZROWS):
        pltpu.sync_copy(acc.at[pl.ds(r0 + j * ZROWS, ZROWS)], zbuf)
        pltpu.sync_copy(zbuf, degp_hbm.at[c, pl.ds(r0 + j * ZROWS, ZROWS)])


def _sc_degree(src, zeros16, ones16):
    f = pl.kernel(
        _sc_degree_body,
        out_type=jax.ShapeDtypeStruct((NC, N, 16), _f32),
        mesh=_MESH,
        scratch_types=[
            pltpu.VMEM_SHARED((N, 16), _f32),
            pltpu.VMEM((ZROWS, 16), _f32),
            pltpu.VMEM((CHUNK, 16), _f32),
            pltpu.VMEM((CHUNK,), jnp.int32),
            pltpu.SemaphoreType.DMA,
        ],
        compiler_params=pltpu.CompilerParams(use_tc_tiling_on_sc=False),
        name="sc_degree",
    )
    return f(src, zeros16, ones16)


def _sc_round_body(vlo_hbm, vhi_hbm, src_hbm, dst_hbm, zeros_hbm, out_hbm,
                   acc, zbuf, sidx, didx, rows, gsem):
    c = lax.axis_index("c")
    s = lax.axis_index("s")
    r0 = s * RPT
    pltpu.sync_copy(zeros_hbm, zbuf)
    for j in range(RPT // ZROWS):
        pltpu.sync_copy(zbuf, acc.at[pl.ds(r0 + j * ZROWS, ZROWS)])
    plsc.subcore_barrier()

    epw = E // NS
    nchunks = epw // CHUNK
    ebase = s * epw

    def run(vsrc):
        def body(t, carry):
            off = ebase + t * CHUNK
            pltpu.sync_copy(src_hbm.at[pl.ds(off, CHUNK)], sidx)
            pltpu.sync_copy(dst_hbm.at[pl.ds(off, CHUNK)], didx)
            pltpu.async_copy(vsrc.at[sidx], rows.at[0], gsem).wait()
            pltpu.sync_copy(rows.at[0], acc.at[didx], add=True)
            return carry
        lax.fori_loop(0, nchunks, body, 0)

    @pl.when(c == 0)
    def _():
        run(vlo_hbm)

    @pl.when(c == 1)
    def _():
        run(vhi_hbm)

    plsc.subcore_barrier()
    for j in range(RPT // ZROWS):
        pltpu.sync_copy(acc.at[pl.ds(r0 + j * ZROWS, ZROWS)], zbuf)
        pltpu.sync_copy(zbuf, out_hbm.at[c, pl.ds(r0 + j * ZROWS, ZROWS)])


def _sc_round(vlo, vhi, src, dst, zeros_half):
    f = pl.kernel(
        _sc_round_body,
        out_type=jax.ShapeDtypeStruct((NC, N, HALF), _f32),
        mesh=_MESH,
        scratch_types=[
            pltpu.VMEM_SHARED((N, HALF), _f32),
            pltpu.VMEM((ZROWS, HALF), _f32),
            pltpu.VMEM((CHUNK,), jnp.int32),
            pltpu.VMEM((CHUNK,), jnp.int32),
            pltpu.VMEM((1, CHUNK, HALF), _f32),
            pltpu.SemaphoreType.DMA,
        ],
        compiler_params=pltpu.CompilerParams(use_tc_tiling_on_sc=False),
        name="sc_p_round",
    )
    return f(vlo, vhi, src, dst, zeros_half)


_NB = 3968


def _prescale_body(x_ref, degp_ref, xslo_ref, xshi_ref, dinv_ref):
    deg = degp_ref[0, :, 0:1] + degp_ref[1, :, 0:1]
    dinv = jnp.where(deg > 0.0, lax.rsqrt(deg), 0.0)
    xs = x_ref[...] * dinv
    xslo_ref[...] = xs[:, :HALF]
    xshi_ref[...] = xs[:, HALF:]
    dinv_ref[...] = dinv


def _tc_prescale(x, degp):
    grid = (N // _NB,)
    return pl.pallas_call(
        _prescale_body,
        grid=grid,
        in_specs=[
            pl.BlockSpec((_NB, CIN), lambda i: (i, 0)),
            pl.BlockSpec((NC, _NB, 16), lambda i: (0, i, 0)),
        ],
        out_specs=[
            pl.BlockSpec((_NB, HALF), lambda i: (i, 0)),
            pl.BlockSpec((_NB, HALF), lambda i: (i, 0)),
            pl.BlockSpec((_NB, 1), lambda i: (i, 0)),
        ],
        out_shape=[
            jax.ShapeDtypeStruct((N, HALF), _f32),
            jax.ShapeDtypeStruct((N, HALF), _f32),
            jax.ShapeDtypeStruct((N, 1), _f32),
        ],
        name="tc_prescale",
    )(x, degp)


def _mid_body(g1p_ref, dinv_ref, s2lo_ref, s2hi_ref, tx1_ref):
    dinv = dinv_ref[...]
    g1 = jnp.concatenate([g1p_ref[0], g1p_ref[1]], axis=1)
    tx1 = -dinv * g1
    tx1_ref[...] = tx1
    s2 = dinv * tx1
    s2lo_ref[...] = s2[:, :HALF]
    s2hi_ref[...] = s2[:, HALF:]


def _tc_mid(g1p, dinv):
    grid = (N // _NB,)
    return pl.pallas_call(
        _mid_body,
        grid=grid,
        in_specs=[
            pl.BlockSpec((NC, _NB, HALF), lambda i: (0, i, 0)),
            pl.BlockSpec((_NB, 1), lambda i: (i, 0)),
        ],
        out_specs=[
            pl.BlockSpec((_NB, HALF), lambda i: (i, 0)),
            pl.BlockSpec((_NB, HALF), lambda i: (i, 0)),
            pl.BlockSpec((_NB, CIN), lambda i: (i, 0)),
        ],
        out_shape=[
            jax.ShapeDtypeStruct((N, HALF), _f32),
            jax.ShapeDtypeStruct((N, HALF), _f32),
            jax.ShapeDtypeStruct((N, CIN), _f32),
        ],
        name="tc_mid",
    )(g1p, dinv)


def _cheb_body(x_ref, tx1_ref, g2p_ref, dinv_ref, w_ref, b_ref, h_ref):
    x = x_ref[...]
    dinv = dinv_ref[...]
    g2 = jnp.concatenate([g2p_ref[0], g2p_ref[1]], axis=1)
    tx1 = tx1_ref[...]
    tx2 = -2.0 * dinv * g2 - x
    out = jnp.dot(x, w_ref[0], preferred_element_type=_f32)
    out += jnp.dot(tx1, w_ref[1], preferred_element_type=_f32)
    out += jnp.dot(tx2, w_ref[2], preferred_element_type=_f32)
    h_ref[...] = jnp.maximum(out + b_ref[...], 0.0)


def _tc_cheb(x, tx1, g2p, dinv, cheb_W, cheb_b):
    grid = (N // _NB,)
    return pl.pallas_call(
        _cheb_body,
        grid=grid,
        in_specs=[
            pl.BlockSpec((_NB, CIN), lambda i: (i, 0)),
            pl.BlockSpec((_NB, CIN), lambda i: (i, 0)),
            pl.BlockSpec((NC, _NB, HALF), lambda i: (0, i, 0)),
            pl.BlockSpec((_NB, 1), lambda i: (i, 0)),
            pl.BlockSpec((3, CIN, COUT), lambda i: (0, 0, 0)),
            pl.BlockSpec((1, COUT), lambda i: (0, 0)),
        ],
        out_specs=pl.BlockSpec((_NB, COUT), lambda i: (i, 0)),
        out_shape=jax.ShapeDtypeStruct((N, COUT), _f32),
        name="tc_cheb",
    )(x, tx1, g2p, dinv, cheb_W, cheb_b)


_KB = 3968


def _head_body(hb_ref, w1_ref, b1_ref, s1_ref, o1_ref,
               w2_ref, b2_ref, s2_ref, o2_ref, w3_ref, b3_ref,
               out_ref, acc_ref):
    k = pl.program_id(0)
    nk = pl.num_programs(0)

    @pl.when(k == 0)
    def _():
        acc_ref[...] = jnp.zeros_like(acc_ref)

    acc_ref[...] += jnp.dot(hb_ref[...], w1_ref[...],
                            preferred_element_type=_f32)

    @pl.when(k == nk - 1)
    def _():
        z = acc_ref[...] + b1_ref[...]
        z = jnp.maximum(z * s1_ref[...] + o1_ref[...], 0.0)
        z = jnp.dot(z, w2_ref[...], preferred_element_type=_f32) + b2_ref[...]
        z = jnp.maximum(z * s2_ref[...] + o2_ref[...], 0.0)
        out_ref[...] = jnp.dot(z, w3_ref[...],
                               preferred_element_type=_f32) + b3_ref[...]


def _tc_head(hb, fc1_W, b1, s1, o1, fc2_W, b2, s2, o2, fc3_Wp, b3p):
    nk = N // _KB
    return pl.pallas_call(
        _head_body,
        grid=(nk,),
        in_specs=[
            pl.BlockSpec((BATCH, _KB), lambda k: (0, k)),
            pl.BlockSpec((_KB, LIN1), lambda k: (k, 0)),
            pl.BlockSpec((1, LIN1), lambda k: (0, 0)),
            pl.BlockSpec((1, LIN1), lambda k: (0, 0)),
            pl.BlockSpec((1, LIN1), lambda k: (0, 0)),
            pl.BlockSpec((LIN1, LIN2), lambda k: (0, 0)),
            pl.BlockSpec((1, LIN2), lambda k: (0, 0)),
            pl.BlockSpec((1, LIN2), lambda k: (0, 0)),
            pl.BlockSpec((1, LIN2), lambda k: (0, 0)),
            pl.BlockSpec((LIN2, 128), lambda k: (0, 0)),
            pl.BlockSpec((1, 128), lambda k: (0, 0)),
        ],
        out_specs=pl.BlockSpec((BATCH, 128), lambda k: (0, 0)),
        out_shape=jax.ShapeDtypeStruct((BATCH, 128), _f32),
        scratch_shapes=[pltpu.VMEM((BATCH, LIN1), _f32)],
        name="tc_head",
    )(hb, fc1_W, b1, s1, o1, fc2_W, b2, s2, o2, fc3_Wp, b3p)


def kernel(x, edge_index, cheb_W, cheb_b, fc1_W, fc1_b,
           bn1_gamma, bn1_beta, bn1_mean, bn1_var,
           fc2_W, fc2_b, bn2_gamma, bn2_beta, bn2_mean, bn2_var,
           fc3_W, fc3_b):
    src = edge_index[0].astype(jnp.int32)
    dst = edge_index[1].astype(jnp.int32)

    zeros16 = jnp.zeros((ZROWS, 16), _f32)
    ones16 = jnp.ones((CHUNK, 16), _f32)
    zeros_half = jnp.zeros((ZROWS, HALF), _f32)

    degp = _sc_degree(src, zeros16, ones16)
    xslo, xshi, dinv = _tc_prescale(x, degp)
    g1p = _sc_round(xslo, xshi, src, dst, zeros_half)
    s2lo, s2hi, tx1 = _tc_mid(g1p, dinv)
    g2p = _sc_round(s2lo, s2hi, src, dst, zeros_half)
    h = _tc_cheb(x, tx1, g2p, dinv, cheb_W, cheb_b.reshape(1, COUT))

    hb = h.reshape(BATCH, 62 * COUT)

    s1 = (bn1_gamma / jnp.sqrt(bn1_var + EPS)).reshape(1, LIN1)
    o1 = (bn1_beta - bn1_mean * s1[0]).reshape(1, LIN1)
    sc2 = (bn2_gamma / jnp.sqrt(bn2_var + EPS)).reshape(1, LIN2)
    o2 = (bn2_beta - bn2_mean * sc2[0]).reshape(1, LIN2)
    fc3_Wp = jnp.pad(fc3_W, ((0, 0), (0, 128 - fc3_W.shape[1])))
    b3p = jnp.pad(fc3_b, (0, 128 - fc3_b.shape[0])).reshape(1, 128)

    out = _tc_head(hb, fc1_W, fc1_b.reshape(1, LIN1), s1, o1,
                   fc2_W, fc2_b.reshape(1, LIN2), sc2, o2, fc3_Wp, b3p)
    return out[:, :fc3_W.shape[1]]


# speedup vs baseline: 24.6256x; 1.8311x over previous
import jax
import jax.numpy as jnp
from jax import lax
from jax.experimental import pallas as pl
from jax.experimental.pallas import tpu as pltpu
from jax.experimental.pallas import tpu_sc as plsc

N = 15872
E = 507904
CIN = 128
HALF = CIN // 2
COUT = 256
NC, NS = 2, 16
CHUNK = 128
RPT = N // NS
ZROWS = 248
BATCH = 256
LIN1 = 512
LIN2 = 256
EPS = 1e-5

_MESH = plsc.VectorSubcoreMesh(
    core_axis_name="c", subcore_axis_name="s", num_cores=NC, num_subcores=NS)

_f32 = jnp.float32


def _sc_degree_body(src_hbm, zeros_hbm, ones_hbm, degp_hbm,
                    acc, zbuf, ones_v, sidx, gsem):
    c = lax.axis_index("c")
    s = lax.axis_index("s")
    r0 = s * RPT
    pltpu.sync_copy(zeros_hbm, zbuf)
    for j in range(RPT // ZROWS):
        pltpu.sync_copy(zbuf, acc.at[pl.ds(r0 + j * ZROWS, ZROWS)])
    pltpu.sync_copy(ones_hbm, ones_v)
    plsc.subcore_barrier()

    w = s * NC + c
    epw = E // (NC * NS)
    nchunks = epw // CHUNK
    ebase = w * epw

    def body(t, carry):
        off = ebase + t * CHUNK
        pltpu.sync_copy(src_hbm.at[pl.ds(off, CHUNK)], sidx)
        pltpu.sync_copy(ones_v, acc.at[sidx], add=True)
        return carry
    lax.fori_loop(0, nchunks, body, 0)

    plsc.subcore_barrier()
    for j in range(RPT // ZROWS):
        pltpu.sync_copy(acc.at[pl.ds(r0 + j * ZROWS, ZROWS)], zbuf)
        pltpu.sync_copy(zbuf, degp_hbm.at[c, pl.ds(r0 + j * ZROWS, ZROWS)])


def _sc_degree(src, zeros16, ones16):
    f = pl.kernel(
        _sc_degree_body,
        out_type=jax.ShapeDtypeStruct((NC, N, 16), _f32),
        mesh=_MESH,
        scratch_types=[
            pltpu.VMEM_SHARED((N, 16), _f32),
            pltpu.VMEM((ZROWS, 16), _f32),
            pltpu.VMEM((CHUNK, 16), _f32),
            pltpu.VMEM((CHUNK,), jnp.int32),
            pltpu.SemaphoreType.DMA,
        ],
        compiler_params=pltpu.CompilerParams(use_tc_tiling_on_sc=False),
        name="sc_degree",
    )
    return f(src, zeros16, ones16)


NBLK = 4
BCH = 62
PAIRS = BCH // 2


def _sc_round_body(vlo_hbm, vhi_hbm, src4_hbm, dst4_hbm, zeros_hbm, out_hbm,
                   acc, zbuf, sidx, didx, rows0, rows1,
                   gsem0, gsem1, ssem0, ssem1):
    c = lax.axis_index("c")
    s = lax.axis_index("s")
    r0 = s * RPT
    pltpu.sync_copy(zeros_hbm, zbuf)
    for j in range(RPT // ZROWS):
        pltpu.sync_copy(zbuf, acc.at[pl.ds(r0 + j * ZROWS, ZROWS)])
    plsc.subcore_barrier()

    rows = (rows0, rows1)
    gsems = (gsem0, gsem1)
    ssems = (ssem0, ssem1)

    def run(vsrc):
        for blk in range(NBLK):
            pltpu.sync_copy(src4_hbm.at[s, blk], sidx)
            pltpu.sync_copy(dst4_hbm.at[s, blk], didx)

            def pair(p, carry):
                descs = []
                for b in range(2):
                    @pl.when(p > 0)
                    def _(b=b):
                        pltpu.make_async_copy(
                            rows[b], acc.at[didx.at[2 * p + b]],
                            ssems[b]).wait()
                    descs.append(pltpu.async_copy(
                        vsrc.at[sidx.at[2 * p + b]], rows[b], gsems[b]))
                for b in range(2):
                    descs[b].wait()
                    pltpu.async_copy(rows[b], acc.at[didx.at[2 * p + b]],
                                     ssems[b], add=True)
                return carry
            lax.fori_loop(0, PAIRS, pair, 0)

            for b in range(2):
                pltpu.make_async_copy(
                    rows[b], acc.at[didx.at[2 * PAIRS - 2 + b]],
                    ssems[b]).wait()

    @pl.when(c == 0)
    def _():
        run(vlo_hbm)

    @pl.when(c == 1)
    def _():
        run(vhi_hbm)

    plsc.subcore_barrier()
    for j in range(RPT // ZROWS):
        pltpu.sync_copy(acc.at[pl.ds(r0 + j * ZROWS, ZROWS)], zbuf)
        pltpu.sync_copy(zbuf, out_hbm.at[c, pl.ds(r0 + j * ZROWS, ZROWS)])


def _sc_round(vlo, vhi, src, dst, zeros_half):
    src4 = src.reshape(NS, NBLK, BCH, CHUNK)
    dst4 = dst.reshape(NS, NBLK, BCH, CHUNK)
    f = pl.kernel(
        _sc_round_body,
        out_type=jax.ShapeDtypeStruct((NC, N, HALF), _f32),
        mesh=_MESH,
        scratch_types=[
            pltpu.VMEM_SHARED((N, HALF), _f32),
            pltpu.VMEM((ZROWS, HALF), _f32),
            pltpu.VMEM((BCH, CHUNK), jnp.int32),
            pltpu.VMEM((BCH, CHUNK), jnp.int32),
            pltpu.VMEM((CHUNK, HALF), _f32),
            pltpu.VMEM((CHUNK, HALF), _f32),
            pltpu.SemaphoreType.DMA,
            pltpu.SemaphoreType.DMA,
            pltpu.SemaphoreType.DMA,
            pltpu.SemaphoreType.DMA,
        ],
        compiler_params=pltpu.CompilerParams(use_tc_tiling_on_sc=False),
        name="sc_p_round",
    )
    return f(vlo, vhi, src4, dst4, zeros_half)


_NB = 3968


def _prescale_body(x_ref, degp_ref, xslo_ref, xshi_ref, dinv_ref):
    deg = degp_ref[0, :, 0:1] + degp_ref[1, :, 0:1]
    dinv = jnp.where(deg > 0.0, lax.rsqrt(deg), 0.0)
    xs = x_ref[...] * dinv
    xslo_ref[...] = xs[:, :HALF]
    xshi_ref[...] = xs[:, HALF:]
    dinv_ref[...] = dinv


def _tc_prescale(x, degp):
    grid = (N // _NB,)
    return pl.pallas_call(
        _prescale_body,
        grid=grid,
        in_specs=[
            pl.BlockSpec((_NB, CIN), lambda i: (i, 0)),
            pl.BlockSpec((NC, _NB, 16), lambda i: (0, i, 0)),
        ],
        out_specs=[
            pl.BlockSpec((_NB, HALF), lambda i: (i, 0)),
            pl.BlockSpec((_NB, HALF), lambda i: (i, 0)),
            pl.BlockSpec((_NB, 1), lambda i: (i, 0)),
        ],
        out_shape=[
            jax.ShapeDtypeStruct((N, HALF), _f32),
            jax.ShapeDtypeStruct((N, HALF), _f32),
            jax.ShapeDtypeStruct((N, 1), _f32),
        ],
        name="tc_prescale",
    )(x, degp)


def _mid_body(g1p_ref, dinv_ref, s2lo_ref, s2hi_ref, tx1_ref):
    dinv = dinv_ref[...]
    g1 = jnp.concatenate([g1p_ref[0], g1p_ref[1]], axis=1)
    tx1 = -dinv * g1
    tx1_ref[...] = tx1
    s2 = dinv * tx1
    s2lo_ref[...] = s2[:, :HALF]
    s2hi_ref[...] = s2[:, HALF:]


def _tc_mid(g1p, dinv):
    grid = (N // _NB,)
    return pl.pallas_call(
        _mid_body,
        grid=grid,
        in_specs=[
            pl.BlockSpec((NC, _NB, HALF), lambda i: (0, i, 0)),
            pl.BlockSpec((_NB, 1), lambda i: (i, 0)),
        ],
        out_specs=[
            pl.BlockSpec((_NB, HALF), lambda i: (i, 0)),
            pl.BlockSpec((_NB, HALF), lambda i: (i, 0)),
            pl.BlockSpec((_NB, CIN), lambda i: (i, 0)),
        ],
        out_shape=[
            jax.ShapeDtypeStruct((N, HALF), _f32),
            jax.ShapeDtypeStruct((N, HALF), _f32),
            jax.ShapeDtypeStruct((N, CIN), _f32),
        ],
        name="tc_mid",
    )(g1p, dinv)


def _cheb_body(x_ref, tx1_ref, g2p_ref, dinv_ref, w_ref, b_ref, h_ref):
    x = x_ref[...]
    dinv = dinv_ref[...]
    g2 = jnp.concatenate([g2p_ref[0], g2p_ref[1]], axis=1)
    tx1 = tx1_ref[...]
    tx2 = -2.0 * dinv * g2 - x
    out = jnp.dot(x, w_ref[0], preferred_element_type=_f32)
    out += jnp.dot(tx1, w_ref[1], preferred_element_type=_f32)
    out += jnp.dot(tx2, w_ref[2], preferred_element_type=_f32)
    h_ref[...] = jnp.maximum(out + b_ref[...], 0.0)


def _tc_cheb(x, tx1, g2p, dinv, cheb_W, cheb_b):
    grid = (N // _NB,)
    return pl.pallas_call(
        _cheb_body,
        grid=grid,
        in_specs=[
            pl.BlockSpec((_NB, CIN), lambda i: (i, 0)),
            pl.BlockSpec((_NB, CIN), lambda i: (i, 0)),
            pl.BlockSpec((NC, _NB, HALF), lambda i: (0, i, 0)),
            pl.BlockSpec((_NB, 1), lambda i: (i, 0)),
            pl.BlockSpec((3, CIN, COUT), lambda i: (0, 0, 0)),
            pl.BlockSpec((1, COUT), lambda i: (0, 0)),
        ],
        out_specs=pl.BlockSpec((_NB, COUT), lambda i: (i, 0)),
        out_shape=jax.ShapeDtypeStruct((N, COUT), _f32),
        name="tc_cheb",
    )(x, tx1, g2p, dinv, cheb_W, cheb_b)


_KB = 3968


def _head_body(hb_ref, w1_ref, b1_ref, s1_ref, o1_ref,
               w2_ref, b2_ref, s2_ref, o2_ref, w3_ref, b3_ref,
               out_ref, acc_ref):
    k = pl.program_id(0)
    nk = pl.num_programs(0)

    @pl.when(k == 0)
    def _():
        acc_ref[...] = jnp.zeros_like(acc_ref)

    acc_ref[...] += jnp.dot(hb_ref[...], w1_ref[...],
                            preferred_element_type=_f32)

    @pl.when(k == nk - 1)
    def _():
        z = acc_ref[...] + b1_ref[...]
        z = jnp.maximum(z * s1_ref[...] + o1_ref[...], 0.0)
        z = jnp.dot(z, w2_ref[...], preferred_element_type=_f32) + b2_ref[...]
        z = jnp.maximum(z * s2_ref[...] + o2_ref[...], 0.0)
        out_ref[...] = jnp.dot(z, w3_ref[...],
                               preferred_element_type=_f32) + b3_ref[...]


def _tc_head(hb, fc1_W, b1, s1, o1, fc2_W, b2, s2, o2, fc3_Wp, b3p):
    nk = N // _KB
    return pl.pallas_call(
        _head_body,
        grid=(nk,),
        in_specs=[
            pl.BlockSpec((BATCH, _KB), lambda k: (0, k)),
            pl.BlockSpec((_KB, LIN1), lambda k: (k, 0)),
            pl.BlockSpec((1, LIN1), lambda k: (0, 0)),
            pl.BlockSpec((1, LIN1), lambda k: (0, 0)),
            pl.BlockSpec((1, LIN1), lambda k: (0, 0)),
            pl.BlockSpec((LIN1, LIN2), lambda k: (0, 0)),
            pl.BlockSpec((1, LIN2), lambda k: (0, 0)),
            pl.BlockSpec((1, LIN2), lambda k: (0, 0)),
            pl.BlockSpec((1, LIN2), lambda k: (0, 0)),
            pl.BlockSpec((LIN2, 128), lambda k: (0, 0)),
            pl.BlockSpec((1, 128), lambda k: (0, 0)),
        ],
        out_specs=pl.BlockSpec((BATCH, 128), lambda k: (0, 0)),
        out_shape=jax.ShapeDtypeStruct((BATCH, 128), _f32),
        scratch_shapes=[pltpu.VMEM((BATCH, LIN1), _f32)],
        name="tc_head",
    )(hb, fc1_W, b1, s1, o1, fc2_W, b2, s2, o2, fc3_Wp, b3p)


def kernel(x, edge_index, cheb_W, cheb_b, fc1_W, fc1_b,
           bn1_gamma, bn1_beta, bn1_mean, bn1_var,
           fc2_W, fc2_b, bn2_gamma, bn2_beta, bn2_mean, bn2_var,
           fc3_W, fc3_b):
    src = edge_index[0].astype(jnp.int32)
    dst = edge_index[1].astype(jnp.int32)

    zeros16 = jnp.zeros((ZROWS, 16), _f32)
    ones16 = jnp.ones((CHUNK, 16), _f32)
    zeros_half = jnp.zeros((ZROWS, HALF), _f32)

    degp = _sc_degree(src, zeros16, ones16)
    xslo, xshi, dinv = _tc_prescale(x, degp)
    g1p = _sc_round(xslo, xshi, src, dst, zeros_half)
    s2lo, s2hi, tx1 = _tc_mid(g1p, dinv)
    g2p = _sc_round(s2lo, s2hi, src, dst, zeros_half)
    h = _tc_cheb(x, tx1, g2p, dinv, cheb_W, cheb_b.reshape(1, COUT))

    hb = h.reshape(BATCH, 62 * COUT)

    s1 = (bn1_gamma / jnp.sqrt(bn1_var + EPS)).reshape(1, LIN1)
    o1 = (bn1_beta - bn1_mean * s1[0]).reshape(1, LIN1)
    sc2 = (bn2_gamma / jnp.sqrt(bn2_var + EPS)).reshape(1, LIN2)
    o2 = (bn2_beta - bn2_mean * sc2[0]).reshape(1, LIN2)
    fc3_Wp = jnp.pad(fc3_W, ((0, 0), (0, 128 - fc3_W.shape[1])))
    b3p = jnp.pad(fc3_b, (0, 128 - fc3_b.shape[0])).reshape(1, 128)

    out = _tc_head(hb, fc1_W, fc1_b.reshape(1, LIN1), s1, o1,
                   fc2_W, fc2_b.reshape(1, LIN2), sc2, o2, fc3_Wp, b3p)
    return out[:, :fc3_W.shape[1]]


# speedup vs baseline: 34.0776x; 1.3838x over previous
import jax
import jax.numpy as jnp
from jax import lax
from jax.experimental import pallas as pl
from jax.experimental.pallas import tpu as pltpu
from jax.experimental.pallas import tpu_sc as plsc

N = 15872
E = 507904
CIN = 128
HALF = CIN // 2
COUT = 256
NC, NS = 2, 16
CHUNK = 128
RPT = N // NS
ZROWS = 248
QROWS = 62
BATCH = 256
LIN1 = 512
LIN2 = 256
EPS = 1e-5

_MESH = plsc.VectorSubcoreMesh(
    core_axis_name="c", subcore_axis_name="s", num_cores=NC, num_subcores=NS)

_f32 = jnp.float32


def _sc_degree_body(src4_hbm, zeros_hbm, ones_hbm, degp_hbm,
                    acc, zbuf, ones_v, sidx, ssem):
    c = lax.axis_index("c")
    s = lax.axis_index("s")
    r0 = s * RPT
    pltpu.sync_copy(zeros_hbm, zbuf)
    for j in range(RPT // ZROWS):
        pltpu.sync_copy(zbuf, acc.at[pl.ds(r0 + j * ZROWS, ZROWS)])
    pltpu.sync_copy(ones_hbm, ones_v)
    plsc.subcore_barrier()

    w = s * NC + c
    LAG = 8

    for blk in range(2):
        pltpu.sync_copy(src4_hbm.at[w, blk], sidx)

        def body(t, carry):
            pltpu.async_copy(ones_v, acc.at[sidx.at[t]], ssem, add=True)

            @pl.when(t >= LAG)
            def _():
                pltpu.make_async_copy(ones_v, acc.at[sidx.at[t]],
                                      ssem).wait()
            return carry
        lax.fori_loop(0, BCH, body, 0)
        for _ in range(LAG):
            pltpu.make_async_copy(ones_v, acc.at[sidx.at[0]], ssem).wait()

    plsc.subcore_barrier()
    for j in range(RPT // ZROWS):
        pltpu.sync_copy(acc.at[pl.ds(r0 + j * ZROWS, ZROWS)], zbuf)
        pltpu.sync_copy(zbuf, degp_hbm.at[c, pl.ds(r0 + j * ZROWS, ZROWS)])


def _sc_degree(src, zeros16, ones16):
    f = pl.kernel(
        _sc_degree_body,
        out_type=jax.ShapeDtypeStruct((NC, N, 16), _f32),
        mesh=_MESH,
        scratch_types=[
            pltpu.VMEM_SHARED((N, 16), _f32),
            pltpu.VMEM((ZROWS, 16), _f32),
            pltpu.VMEM((CHUNK, 16), _f32),
            pltpu.VMEM((BCH, CHUNK), jnp.int32),
            pltpu.SemaphoreType.DMA,
        ],
        compiler_params=pltpu.CompilerParams(use_tc_tiling_on_sc=False),
        name="sc_degree",
    )
    return f(src.reshape(NC * NS, 2, BCH, CHUNK), zeros16, ones16)


NBLK = 4
BCH = 62
PAIRS = BCH // 2


def _sc_round_body(vlo_hbm, vhi_hbm, src4_hbm, dst4_hbm, zeros_hbm, out_hbm,
                   acc, zbuf, sidx, didx, rows0, rows1, rows2, rows3,
                   gsem0, gsem1, gsem2, gsem3, ssem0, ssem1, ssem2, ssem3):
    c = lax.axis_index("c")
    s = lax.axis_index("s")
    r0 = s * RPT
    pltpu.sync_copy(zeros_hbm, zbuf)
    for j in range(RPT // QROWS):
        pltpu.sync_copy(zbuf, acc.at[pl.ds(r0 + j * QROWS, QROWS)])
    plsc.subcore_barrier()

    rows = (rows0, rows1, rows2, rows3)
    gsems = (gsem0, gsem1, gsem2, gsem3)
    ssems = (ssem0, ssem1, ssem2, ssem3)

    NQ = 15

    def run(vsrc):
        for blk in range(NBLK):
            pltpu.sync_copy(src4_hbm.at[s, blk], sidx)
            pltpu.sync_copy(dst4_hbm.at[s, blk], didx)

            def quad(p, carry):
                descs = []
                for b in range(4):
                    @pl.when(p > 0)
                    def _(b=b):
                        pltpu.make_async_copy(
                            rows[b], acc.at[didx.at[4 * p + b]],
                            ssems[b]).wait()
                    descs.append(pltpu.async_copy(
                        vsrc.at[sidx.at[4 * p + b]], rows[b], gsems[b]))
                for b in range(4):
                    descs[b].wait()
                    pltpu.async_copy(rows[b], acc.at[didx.at[4 * p + b]],
                                     ssems[b], add=True)
                return carry
            lax.fori_loop(0, NQ, quad, 0)

            descs = []
            for b in range(2):
                pltpu.make_async_copy(
                    rows[b], acc.at[didx.at[4 * NQ + b]], ssems[b]).wait()
                descs.append(pltpu.async_copy(
                    vsrc.at[sidx.at[4 * NQ + b]], rows[b], gsems[b]))
            for b in range(2):
                descs[b].wait()
                pltpu.async_copy(rows[b], acc.at[didx.at[4 * NQ + b]],
                                 ssems[b], add=True)

            for b in range(4):
                pltpu.make_async_copy(
                    rows[b], acc.at[didx.at[b]], ssems[b]).wait()

    @pl.when(c == 0)
    def _():
        run(vlo_hbm)

    @pl.when(c == 1)
    def _():
        run(vhi_hbm)

    plsc.subcore_barrier()
    for j in range(RPT // QROWS):
        pltpu.sync_copy(acc.at[pl.ds(r0 + j * QROWS, QROWS)], zbuf)
        pltpu.sync_copy(zbuf, out_hbm.at[c, pl.ds(r0 + j * QROWS, QROWS)])


def _sc_round(vlo, vhi, src, dst, zeros_half):
    src4 = src.reshape(NS, NBLK, BCH, CHUNK)
    dst4 = dst.reshape(NS, NBLK, BCH, CHUNK)
    f = pl.kernel(
        _sc_round_body,
        out_type=jax.ShapeDtypeStruct((NC, N, HALF), _f32),
        mesh=_MESH,
        scratch_types=[
            pltpu.VMEM_SHARED((N, HALF), _f32),
            pltpu.VMEM((QROWS, HALF), _f32),
            pltpu.VMEM((BCH, CHUNK), jnp.int32),
            pltpu.VMEM((BCH, CHUNK), jnp.int32),
            pltpu.VMEM((CHUNK, HALF), _f32),
            pltpu.VMEM((CHUNK, HALF), _f32),
            pltpu.VMEM((CHUNK, HALF), _f32),
            pltpu.VMEM((CHUNK, HALF), _f32),
        ] + [pltpu.SemaphoreType.DMA] * 8,
        compiler_params=pltpu.CompilerParams(use_tc_tiling_on_sc=False),
        name="sc_p_round",
    )
    return f(vlo, vhi, src4, dst4, zeros_half)


_NB = 3968


def _prescale_body(x_ref, degp_ref, xslo_ref, xshi_ref, dinv_ref):
    deg = degp_ref[0, :, 0:1] + degp_ref[1, :, 0:1]
    dinv = jnp.where(deg > 0.0, lax.rsqrt(deg), 0.0)
    xs = x_ref[...] * dinv
    xslo_ref[...] = xs[:, :HALF]
    xshi_ref[...] = xs[:, HALF:]
    dinv_ref[...] = dinv


def _tc_prescale(x, degp):
    grid = (N // _NB,)
    return pl.pallas_call(
        _prescale_body,
        grid=grid,
        in_specs=[
            pl.BlockSpec((_NB, CIN), lambda i: (i, 0)),
            pl.BlockSpec((NC, _NB, 16), lambda i: (0, i, 0)),
        ],
        out_specs=[
            pl.BlockSpec((_NB, HALF), lambda i: (i, 0)),
            pl.BlockSpec((_NB, HALF), lambda i: (i, 0)),
            pl.BlockSpec((_NB, 1), lambda i: (i, 0)),
        ],
        out_shape=[
            jax.ShapeDtypeStruct((N, HALF), _f32),
            jax.ShapeDtypeStruct((N, HALF), _f32),
            jax.ShapeDtypeStruct((N, 1), _f32),
        ],
        name="tc_prescale",
    )(x, degp)


def _mid_body(g1p_ref, dinv_ref, s2lo_ref, s2hi_ref, tx1_ref):
    dinv = dinv_ref[...]
    g1 = jnp.concatenate([g1p_ref[0], g1p_ref[1]], axis=1)
    tx1 = -dinv * g1
    tx1_ref[...] = tx1
    s2 = dinv * tx1
    s2lo_ref[...] = s2[:, :HALF]
    s2hi_ref[...] = s2[:, HALF:]


def _tc_mid(g1p, dinv):
    grid = (N // _NB,)
    return pl.pallas_call(
        _mid_body,
        grid=grid,
        in_specs=[
            pl.BlockSpec((NC, _NB, HALF), lambda i: (0, i, 0)),
            pl.BlockSpec((_NB, 1), lambda i: (i, 0)),
        ],
        out_specs=[
            pl.BlockSpec((_NB, HALF), lambda i: (i, 0)),
            pl.BlockSpec((_NB, HALF), lambda i: (i, 0)),
            pl.BlockSpec((_NB, CIN), lambda i: (i, 0)),
        ],
        out_shape=[
            jax.ShapeDtypeStruct((N, HALF), _f32),
            jax.ShapeDtypeStruct((N, HALF), _f32),
            jax.ShapeDtypeStruct((N, CIN), _f32),
        ],
        name="tc_mid",
    )(g1p, dinv)


def _cheb_body(x_ref, tx1_ref, g2p_ref, dinv_ref, w_ref, b_ref, h_ref):
    x = x_ref[...]
    dinv = dinv_ref[...]
    g2 = jnp.concatenate([g2p_ref[0], g2p_ref[1]], axis=1)
    tx1 = tx1_ref[...]
    tx2 = -2.0 * dinv * g2 - x
    out = jnp.dot(x, w_ref[0], preferred_element_type=_f32)
    out += jnp.dot(tx1, w_ref[1], preferred_element_type=_f32)
    out += jnp.dot(tx2, w_ref[2], preferred_element_type=_f32)
    h_ref[...] = jnp.maximum(out + b_ref[...], 0.0)


def _tc_cheb(x, tx1, g2p, dinv, cheb_W, cheb_b):
    grid = (N // _NB,)
    return pl.pallas_call(
        _cheb_body,
        grid=grid,
        in_specs=[
            pl.BlockSpec((_NB, CIN), lambda i: (i, 0)),
            pl.BlockSpec((_NB, CIN), lambda i: (i, 0)),
            pl.BlockSpec((NC, _NB, HALF), lambda i: (0, i, 0)),
            pl.BlockSpec((_NB, 1), lambda i: (i, 0)),
            pl.BlockSpec((3, CIN, COUT), lambda i: (0, 0, 0)),
            pl.BlockSpec((1, COUT), lambda i: (0, 0)),
        ],
        out_specs=pl.BlockSpec((_NB, COUT), lambda i: (i, 0)),
        out_shape=jax.ShapeDtypeStruct((N, COUT), _f32),
        name="tc_cheb",
    )(x, tx1, g2p, dinv, cheb_W, cheb_b)


_KB = 3968


def _head_body(hb_ref, w1_ref, b1_ref, s1_ref, o1_ref,
               w2_ref, b2_ref, s2_ref, o2_ref, w3_ref, b3_ref,
               out_ref, acc_ref):
    k = pl.program_id(0)
    nk = pl.num_programs(0)

    @pl.when(k == 0)
    def _():
        acc_ref[...] = jnp.zeros_like(acc_ref)

    acc_ref[...] += jnp.dot(hb_ref[...], w1_ref[...],
                            preferred_element_type=_f32)

    @pl.when(k == nk - 1)
    def _():
        z = acc_ref[...] + b1_ref[...]
        z = jnp.maximum(z * s1_ref[...] + o1_ref[...], 0.0)
        z = jnp.dot(z, w2_ref[...], preferred_element_type=_f32) + b2_ref[...]
        z = jnp.maximum(z * s2_ref[...] + o2_ref[...], 0.0)
        out_ref[...] = jnp.dot(z, w3_ref[...],
                               preferred_element_type=_f32) + b3_ref[...]


def _tc_head(hb, fc1_W, b1, s1, o1, fc2_W, b2, s2, o2, fc3_Wp, b3p):
    nk = N // _KB
    return pl.pallas_call(
        _head_body,
        grid=(nk,),
        in_specs=[
            pl.BlockSpec((BATCH, _KB), lambda k: (0, k)),
            pl.BlockSpec((_KB, LIN1), lambda k: (k, 0)),
            pl.BlockSpec((1, LIN1), lambda k: (0, 0)),
            pl.BlockSpec((1, LIN1), lambda k: (0, 0)),
            pl.BlockSpec((1, LIN1), lambda k: (0, 0)),
            pl.BlockSpec((LIN1, LIN2), lambda k: (0, 0)),
            pl.BlockSpec((1, LIN2), lambda k: (0, 0)),
            pl.BlockSpec((1, LIN2), lambda k: (0, 0)),
            pl.BlockSpec((1, LIN2), lambda k: (0, 0)),
            pl.BlockSpec((LIN2, 128), lambda k: (0, 0)),
            pl.BlockSpec((1, 128), lambda k: (0, 0)),
        ],
        out_specs=pl.BlockSpec((BATCH, 128), lambda k: (0, 0)),
        out_shape=jax.ShapeDtypeStruct((BATCH, 128), _f32),
        scratch_shapes=[pltpu.VMEM((BATCH, LIN1), _f32)],
        name="tc_head",
    )(hb, fc1_W, b1, s1, o1, fc2_W, b2, s2, o2, fc3_Wp, b3p)


def kernel(x, edge_index, cheb_W, cheb_b, fc1_W, fc1_b,
           bn1_gamma, bn1_beta, bn1_mean, bn1_var,
           fc2_W, fc2_b, bn2_gamma, bn2_beta, bn2_mean, bn2_var,
           fc3_W, fc3_b):
    src = edge_index[0].astype(jnp.int32)
    dst = edge_index[1].astype(jnp.int32)

    zeros16 = jnp.zeros((ZROWS, 16), _f32)
    ones16 = jnp.ones((CHUNK, 16), _f32)
    zeros_half = jnp.zeros((QROWS, HALF), _f32)

    degp = _sc_degree(src, zeros16, ones16)
    xslo, xshi, dinv = _tc_prescale(x, degp)
    g1p = _sc_round(xslo, xshi, src, dst, zeros_half)
    s2lo, s2hi, tx1 = _tc_mid(g1p, dinv)
    g2p = _sc_round(s2lo, s2hi, src, dst, zeros_half)
    h = _tc_cheb(x, tx1, g2p, dinv, cheb_W, cheb_b.reshape(1, COUT))

    hb = h.reshape(BATCH, 62 * COUT)

    s1 = (bn1_gamma / jnp.sqrt(bn1_var + EPS)).reshape(1, LIN1)
    o1 = (bn1_beta - bn1_mean * s1[0]).reshape(1, LIN1)
    sc2 = (bn2_gamma / jnp.sqrt(bn2_var + EPS)).reshape(1, LIN2)
    o2 = (bn2_beta - bn2_mean * sc2[0]).reshape(1, LIN2)
    fc3_Wp = jnp.pad(fc3_W, ((0, 0), (0, 128 - fc3_W.shape[1])))
    b3p = jnp.pad(fc3_b, (0, 128 - fc3_b.shape[0])).reshape(1, 128)

    out = _tc_head(hb, fc1_W, fc1_b.reshape(1, LIN1), s1, o1,
                   fc2_W, fc2_b.reshape(1, LIN2), sc2, o2, fc3_Wp, b3p)
    return out[:, :fc3_W.shape[1]]


# speedup vs baseline: 34.0956x; 1.0005x over previous
import jax
import jax.numpy as jnp
from jax import lax
from jax.experimental import pallas as pl
from jax.experimental.pallas import tpu as pltpu
from jax.experimental.pallas import tpu_sc as plsc

N = 15872
E = 507904
CIN = 128
HALF = CIN // 2
COUT = 256
NC, NS = 2, 16
CHUNK = 128
RPT = N // NS
ZROWS = 248
QROWS = 62
BATCH = 256
LIN1 = 512
LIN2 = 256
EPS = 1e-5

_MESH = plsc.VectorSubcoreMesh(
    core_axis_name="c", subcore_axis_name="s", num_cores=NC, num_subcores=NS)

_f32 = jnp.float32


def _sc_degree_body(src4_hbm, zeros_hbm, ones_hbm, degp_hbm,
                    acc, ones_v, sidx, ssem):
    c = lax.axis_index("c")
    s = lax.axis_index("s")
    r0 = s * RPT
    pltpu.sync_copy(zeros_hbm, acc.at[pl.ds(r0, RPT)])
    pltpu.sync_copy(ones_hbm, ones_v)
    plsc.subcore_barrier()

    w = s * NC + c
    LAG = 8

    for blk in range(2):
        pltpu.sync_copy(src4_hbm.at[w, blk], sidx)

        def body(t, carry):
            pltpu.async_copy(ones_v, acc.at[sidx.at[t]], ssem, add=True)

            @pl.when(t >= LAG)
            def _():
                pltpu.make_async_copy(ones_v, acc.at[sidx.at[t]],
                                      ssem).wait()
            return carry
        lax.fori_loop(0, BCH, body, 0)
        for _ in range(LAG):
            pltpu.make_async_copy(ones_v, acc.at[sidx.at[0]], ssem).wait()

    plsc.subcore_barrier()
    pltpu.sync_copy(acc.at[pl.ds(r0, RPT)], degp_hbm.at[c, pl.ds(r0, RPT)])


def _sc_degree(src, zeros16, ones16):
    f = pl.kernel(
        _sc_degree_body,
        out_type=jax.ShapeDtypeStruct((NC, N, 16), _f32),
        mesh=_MESH,
        scratch_types=[
            pltpu.VMEM_SHARED((N, 16), _f32),
            pltpu.VMEM((CHUNK, 16), _f32),
            pltpu.VMEM((BCH, CHUNK), jnp.int32),
            pltpu.SemaphoreType.DMA,
        ],
        compiler_params=pltpu.CompilerParams(use_tc_tiling_on_sc=False),
        name="sc_degree",
    )
    return f(src.reshape(NC * NS, 2, BCH, CHUNK), zeros16, ones16)


NBLK = 4
BCH = 62
PAIRS = BCH // 2


def _sc_round_body(vlo_hbm, vhi_hbm, src4_hbm, dst4_hbm, zeros_hbm, out_hbm,
                   acc, sidx, didx, rows0, rows1, rows2, rows3,
                   gsem0, gsem1, gsem2, gsem3, ssem0, ssem1, ssem2, ssem3):
    c = lax.axis_index("c")
    s = lax.axis_index("s")
    r0 = s * RPT
    pltpu.sync_copy(zeros_hbm, acc.at[pl.ds(r0, RPT)])
    plsc.subcore_barrier()

    rows = (rows0, rows1, rows2, rows3)
    gsems = (gsem0, gsem1, gsem2, gsem3)
    ssems = (ssem0, ssem1, ssem2, ssem3)

    NQ = 15

    def run(vsrc):
        for blk in range(NBLK):
            pltpu.sync_copy(src4_hbm.at[s, blk], sidx)
            pltpu.sync_copy(dst4_hbm.at[s, blk], didx)

            def quad(p, carry):
                descs = []
                for b in range(4):
                    @pl.when(p > 0)
                    def _(b=b):
                        pltpu.make_async_copy(
                            rows[b], acc.at[didx.at[4 * p + b]],
                            ssems[b]).wait()
                    descs.append(pltpu.async_copy(
                        vsrc.at[sidx.at[4 * p + b]], rows[b], gsems[b]))
                for b in range(4):
                    descs[b].wait()
                    pltpu.async_copy(rows[b], acc.at[didx.at[4 * p + b]],
                                     ssems[b], add=True)
                return carry
            lax.fori_loop(0, NQ, quad, 0)

            descs = []
            for b in range(2):
                pltpu.make_async_copy(
                    rows[b], acc.at[didx.at[4 * NQ + b]], ssems[b]).wait()
                descs.append(pltpu.async_copy(
                    vsrc.at[sidx.at[4 * NQ + b]], rows[b], gsems[b]))
            for b in range(2):
                descs[b].wait()
                pltpu.async_copy(rows[b], acc.at[didx.at[4 * NQ + b]],
                                 ssems[b], add=True)

            for b in range(4):
                pltpu.make_async_copy(
                    rows[b], acc.at[didx.at[b]], ssems[b]).wait()

    @pl.when(c == 0)
    def _():
        run(vlo_hbm)

    @pl.when(c == 1)
    def _():
        run(vhi_hbm)

    plsc.subcore_barrier()
    pltpu.sync_copy(acc.at[pl.ds(r0, RPT)], out_hbm.at[c, pl.ds(r0, RPT)])


def _sc_round(vlo, vhi, src, dst, zeros_half):
    src4 = src.reshape(NS, NBLK, BCH, CHUNK)
    dst4 = dst.reshape(NS, NBLK, BCH, CHUNK)
    f = pl.kernel(
        _sc_round_body,
        out_type=jax.ShapeDtypeStruct((NC, N, HALF), _f32),
        mesh=_MESH,
        scratch_types=[
            pltpu.VMEM_SHARED((N, HALF), _f32),
            pltpu.VMEM((BCH, CHUNK), jnp.int32),
            pltpu.VMEM((BCH, CHUNK), jnp.int32),
            pltpu.VMEM((CHUNK, HALF), _f32),
            pltpu.VMEM((CHUNK, HALF), _f32),
            pltpu.VMEM((CHUNK, HALF), _f32),
            pltpu.VMEM((CHUNK, HALF), _f32),
        ] + [pltpu.SemaphoreType.DMA] * 8,
        compiler_params=pltpu.CompilerParams(use_tc_tiling_on_sc=False),
        name="sc_p_round",
    )
    return f(vlo, vhi, src4, dst4, zeros_half)


_NB = 3968


def _prescale_body(x_ref, degp_ref, xslo_ref, xshi_ref, dinv_ref):
    deg = degp_ref[0, :, 0:1] + degp_ref[1, :, 0:1]
    dinv = jnp.where(deg > 0.0, lax.rsqrt(deg), 0.0)
    xs = x_ref[...] * dinv
    xslo_ref[...] = xs[:, :HALF]
    xshi_ref[...] = xs[:, HALF:]
    dinv_ref[...] = dinv


def _tc_prescale(x, degp):
    grid = (N // _NB,)
    return pl.pallas_call(
        _prescale_body,
        grid=grid,
        in_specs=[
            pl.BlockSpec((_NB, CIN), lambda i: (i, 0)),
            pl.BlockSpec((NC, _NB, 16), lambda i: (0, i, 0)),
        ],
        out_specs=[
            pl.BlockSpec((_NB, HALF), lambda i: (i, 0)),
            pl.BlockSpec((_NB, HALF), lambda i: (i, 0)),
            pl.BlockSpec((_NB, 1), lambda i: (i, 0)),
        ],
        out_shape=[
            jax.ShapeDtypeStruct((N, HALF), _f32),
            jax.ShapeDtypeStruct((N, HALF), _f32),
            jax.ShapeDtypeStruct((N, 1), _f32),
        ],
        name="tc_prescale",
    )(x, degp)


def _mid_body(g1p_ref, dinv_ref, s2lo_ref, s2hi_ref, tx1_ref):
    dinv = dinv_ref[...]
    g1 = jnp.concatenate([g1p_ref[0], g1p_ref[1]], axis=1)
    tx1 = -dinv * g1
    tx1_ref[...] = tx1
    s2 = dinv * tx1
    s2lo_ref[...] = s2[:, :HALF]
    s2hi_ref[...] = s2[:, HALF:]


def _tc_mid(g1p, dinv):
    grid = (N // _NB,)
    return pl.pallas_call(
        _mid_body,
        grid=grid,
        in_specs=[
            pl.BlockSpec((NC, _NB, HALF), lambda i: (0, i, 0)),
            pl.BlockSpec((_NB, 1), lambda i: (i, 0)),
        ],
        out_specs=[
            pl.BlockSpec((_NB, HALF), lambda i: (i, 0)),
            pl.BlockSpec((_NB, HALF), lambda i: (i, 0)),
            pl.BlockSpec((_NB, CIN), lambda i: (i, 0)),
        ],
        out_shape=[
            jax.ShapeDtypeStruct((N, HALF), _f32),
            jax.ShapeDtypeStruct((N, HALF), _f32),
            jax.ShapeDtypeStruct((N, CIN), _f32),
        ],
        name="tc_mid",
    )(g1p, dinv)


def _cheb_body(x_ref, tx1_ref, g2p_ref, dinv_ref, w_ref, b_ref, h_ref):
    x = x_ref[...]
    dinv = dinv_ref[...]
    g2 = jnp.concatenate([g2p_ref[0], g2p_ref[1]], axis=1)
    tx1 = tx1_ref[...]
    tx2 = -2.0 * dinv * g2 - x
    out = jnp.dot(x, w_ref[0], preferred_element_type=_f32)
    out += jnp.dot(tx1, w_ref[1], preferred_element_type=_f32)
    out += jnp.dot(tx2, w_ref[2], preferred_element_type=_f32)
    h_ref[...] = jnp.maximum(out + b_ref[...], 0.0)


def _tc_cheb(x, tx1, g2p, dinv, cheb_W, cheb_b):
    grid = (N // _NB,)
    return pl.pallas_call(
        _cheb_body,
        grid=grid,
        in_specs=[
            pl.BlockSpec((_NB, CIN), lambda i: (i, 0)),
            pl.BlockSpec((_NB, CIN), lambda i: (i, 0)),
            pl.BlockSpec((NC, _NB, HALF), lambda i: (0, i, 0)),
            pl.BlockSpec((_NB, 1), lambda i: (i, 0)),
            pl.BlockSpec((3, CIN, COUT), lambda i: (0, 0, 0)),
            pl.BlockSpec((1, COUT), lambda i: (0, 0)),
        ],
        out_specs=pl.BlockSpec((_NB, COUT), lambda i: (i, 0)),
        out_shape=jax.ShapeDtypeStruct((N, COUT), _f32),
        name="tc_cheb",
    )(x, tx1, g2p, dinv, cheb_W, cheb_b)


_KB = 3968


def _head_body(hb_ref, w1_ref, b1_ref, s1_ref, o1_ref,
               w2_ref, b2_ref, s2_ref, o2_ref, w3_ref, b3_ref,
               out_ref, acc_ref):
    k = pl.program_id(0)
    nk = pl.num_programs(0)

    @pl.when(k == 0)
    def _():
        acc_ref[...] = jnp.zeros_like(acc_ref)

    acc_ref[...] += jnp.dot(hb_ref[...], w1_ref[...],
                            preferred_element_type=_f32)

    @pl.when(k == nk - 1)
    def _():
        z = acc_ref[...] + b1_ref[...]
        z = jnp.maximum(z * s1_ref[...] + o1_ref[...], 0.0)
        z = jnp.dot(z, w2_ref[...], preferred_element_type=_f32) + b2_ref[...]
        z = jnp.maximum(z * s2_ref[...] + o2_ref[...], 0.0)
        out_ref[...] = jnp.dot(z, w3_ref[...],
                               preferred_element_type=_f32) + b3_ref[...]


def _tc_head(hb, fc1_W, b1, s1, o1, fc2_W, b2, s2, o2, fc3_Wp, b3p):
    nk = N // _KB
    return pl.pallas_call(
        _head_body,
        grid=(nk,),
        in_specs=[
            pl.BlockSpec((BATCH, _KB), lambda k: (0, k)),
            pl.BlockSpec((_KB, LIN1), lambda k: (k, 0)),
            pl.BlockSpec((1, LIN1), lambda k: (0, 0)),
            pl.BlockSpec((1, LIN1), lambda k: (0, 0)),
            pl.BlockSpec((1, LIN1), lambda k: (0, 0)),
            pl.BlockSpec((LIN1, LIN2), lambda k: (0, 0)),
            pl.BlockSpec((1, LIN2), lambda k: (0, 0)),
            pl.BlockSpec((1, LIN2), lambda k: (0, 0)),
            pl.BlockSpec((1, LIN2), lambda k: (0, 0)),
            pl.BlockSpec((LIN2, 128), lambda k: (0, 0)),
            pl.BlockSpec((1, 128), lambda k: (0, 0)),
        ],
        out_specs=pl.BlockSpec((BATCH, 128), lambda k: (0, 0)),
        out_shape=jax.ShapeDtypeStruct((BATCH, 128), _f32),
        scratch_shapes=[pltpu.VMEM((BATCH, LIN1), _f32)],
        name="tc_head",
    )(hb, fc1_W, b1, s1, o1, fc2_W, b2, s2, o2, fc3_Wp, b3p)


def kernel(x, edge_index, cheb_W, cheb_b, fc1_W, fc1_b,
           bn1_gamma, bn1_beta, bn1_mean, bn1_var,
           fc2_W, fc2_b, bn2_gamma, bn2_beta, bn2_mean, bn2_var,
           fc3_W, fc3_b):
    src = edge_index[0].astype(jnp.int32)
    dst = edge_index[1].astype(jnp.int32)

    zeros16 = jnp.zeros((RPT, 16), _f32)
    ones16 = jnp.ones((CHUNK, 16), _f32)
    zeros_half = jnp.zeros((RPT, HALF), _f32)

    degp = _sc_degree(src, zeros16, ones16)
    xslo, xshi, dinv = _tc_prescale(x, degp)
    g1p = _sc_round(xslo, xshi, src, dst, zeros_half)
    s2lo, s2hi, tx1 = _tc_mid(g1p, dinv)
    g2p = _sc_round(s2lo, s2hi, src, dst, zeros_half)
    h = _tc_cheb(x, tx1, g2p, dinv, cheb_W, cheb_b.reshape(1, COUT))

    hb = h.reshape(BATCH, 62 * COUT)

    s1 = (bn1_gamma / jnp.sqrt(bn1_var + EPS)).reshape(1, LIN1)
    o1 = (bn1_beta - bn1_mean * s1[0]).reshape(1, LIN1)
    sc2 = (bn2_gamma / jnp.sqrt(bn2_var + EPS)).reshape(1, LIN2)
    o2 = (bn2_beta - bn2_mean * sc2[0]).reshape(1, LIN2)
    fc3_Wp = jnp.pad(fc3_W, ((0, 0), (0, 128 - fc3_W.shape[1])))
    b3p = jnp.pad(fc3_b, (0, 128 - fc3_b.shape[0])).reshape(1, 128)

    out = _tc_head(hb, fc1_W, fc1_b.reshape(1, LIN1), s1, o1,
                   fc2_W, fc2_b.reshape(1, LIN2), sc2, o2, fc3_Wp, b3p)
    return out[:, :fc3_W.shape[1]]
